# baseline (speedup 1.0000x reference)
import jax
import jax.numpy as jnp
from jax.experimental import pallas as pl

_DIMS = [(128, 256), (256, 128), (128, 64), (64, 32), (32, 2)]
_PAD_DIN = [128, 256, 128, 128, 128]
_PAD_DOUT = [256, 128, 128, 128, 128]
_BN = 1000


def _pad2(a, rows, cols):
    return jnp.pad(a, ((0, rows - a.shape[0]), (0, cols - a.shape[1])))


def _gru_stack_body(x_ref, *args):
    *w_refs, out_ref = args
    T = x_ref.shape[0]
    bn = x_ref.shape[1]
    nl = len(_DIMS)
    hs = [jnp.zeros((bn, dp), jnp.float32) for dp in _PAD_DOUT]
    for t in range(T):
        x = x_ref[t].astype(jnp.bfloat16)
        for i in range(nl):
            wx, wh, whh, b = (r[...] for r in w_refs[4 * i:4 * i + 4])
            dp = _PAD_DOUT[i]
            h = hs[i]
            hb = h.astype(jnp.bfloat16)
            xw = jnp.dot(x, wx, preferred_element_type=jnp.float32) + b
            hw = jnp.dot(hb, wh, preferred_element_type=jnp.float32)
            z = jax.nn.sigmoid(xw[:, :dp] + hw[:, :dp])
            r = jax.nn.sigmoid(xw[:, dp:2 * dp] + hw[:, dp:])
            h_tilde = jnp.tanh(
                xw[:, 2 * dp:]
                + jnp.dot((h * r).astype(jnp.bfloat16), whh,
                          preferred_element_type=jnp.float32))
            h_new = z * h + (1.0 - z) * h_tilde
            if i > 0:
                h_new = jnp.maximum(h_new, 0.0)
            hs[i] = h_new
            x = h_new.astype(jnp.bfloat16)
        out_ref[t] = hs[-1][:, :2]


def kernel(graphs, edge_index, params):
    del edge_index
    T, N, F = graphs.shape
    inputs = [graphs]
    in_specs = [pl.BlockSpec((T, _BN, F), lambda i: (0, i, 0))]
    for i, p in enumerate(params):
        _, dout = _DIMS[i]
        dip, dp = _PAD_DIN[i], _PAD_DOUT[i]
        wx = jnp.concatenate(
            [_pad2(p['Wxz'], dip, dp),
             _pad2(p['Wxr'], dip, dp),
             _pad2(p['Wxh'], dip, dp)], axis=1).astype(jnp.bfloat16)
        wh = jnp.concatenate(
            [_pad2(p['Whz'], dp, dp),
             _pad2(p['Whr'], dp, dp)], axis=1).astype(jnp.bfloat16)
        whh = _pad2(p['Whh'], dp, dp).astype(jnp.bfloat16)
        b = jnp.concatenate([
            jnp.pad(p['bxz'] + p['bhz'], (0, dp - dout)),
            jnp.pad(p['bxr'] + p['bhr'], (0, dp - dout)),
            jnp.pad(p['bxh'] + p['bhh'], (0, dp - dout)),
        ])[None, :]
        inputs += [wx, wh, whh, b]
        in_specs += [
            pl.BlockSpec(wx.shape, lambda i: (0, 0)),
            pl.BlockSpec(wh.shape, lambda i: (0, 0)),
            pl.BlockSpec(whh.shape, lambda i: (0, 0)),
            pl.BlockSpec(b.shape, lambda i: (0, 0)),
        ]
    return pl.pallas_call(
        _gru_stack_body,
        grid=(N // _BN,),
        in_specs=in_specs,
        out_specs=pl.BlockSpec((T, _BN, 2), lambda i: (0, i, 0)),
        out_shape=jax.ShapeDtypeStruct((T, N, 2), jnp.float32),
    )(*inputs)


# speedup vs baseline: 1.0392x; 1.0392x over previous
import jax
import jax.numpy as jnp
from jax.experimental import pallas as pl
from jax.experimental.pallas import tpu as pltpu

_DIMS = [(128, 256), (256, 128), (128, 64), (64, 32), (32, 2)]
_PAD_DIN = [128, 256, 128, 128, 128]
_PAD_DOUT = [256, 128, 128, 128, 128]
_BN = 1000


def _pad2(a, rows, cols):
    return jnp.pad(a, ((0, rows - a.shape[0]), (0, cols - a.shape[1])))


def _gru_stack_body(x_ref, *args):
    *w_refs, out_ref = args
    T = x_ref.shape[0]
    bn = x_ref.shape[1]
    nl = len(_DIMS)
    hs = [jnp.zeros((bn, dp), jnp.float32) for dp in _PAD_DOUT]
    for t in range(T):
        x = x_ref[t]
        for i in range(nl):
            wx, wh, whh, b = (r[...] for r in w_refs[4 * i:4 * i + 4])
            dp = _PAD_DOUT[i]
            h = hs[i]
            xw = jnp.dot(x, wx, preferred_element_type=jnp.float32) + b
            hw = jnp.dot(h, wh, preferred_element_type=jnp.float32)
            z = jax.nn.sigmoid(xw[:, :dp] + hw[:, :dp])
            r = jax.nn.sigmoid(xw[:, dp:2 * dp] + hw[:, dp:])
            h_tilde = jnp.tanh(
                xw[:, 2 * dp:]
                + jnp.dot(h * r, whh, preferred_element_type=jnp.float32))
            h_new = z * h + (1.0 - z) * h_tilde
            if i > 0:
                h_new = jnp.maximum(h_new, 0.0)
            hs[i] = h_new
            x = h_new
        out_ref[t] = hs[-1][:, :2]


def kernel(graphs, edge_index, params):
    del edge_index
    T, N, F = graphs.shape
    inputs = [graphs]
    in_specs = [pl.BlockSpec((T, _BN, F), lambda i: (0, i, 0))]
    for i, p in enumerate(params):
        _, dout = _DIMS[i]
        dip, dp = _PAD_DIN[i], _PAD_DOUT[i]
        wx = jnp.concatenate(
            [_pad2(p['Wxz'], dip, dp),
             _pad2(p['Wxr'], dip, dp),
             _pad2(p['Wxh'], dip, dp)], axis=1)
        wh = jnp.concatenate(
            [_pad2(p['Whz'], dp, dp),
             _pad2(p['Whr'], dp, dp)], axis=1)
        whh = _pad2(p['Whh'], dp, dp)
        b = jnp.concatenate([
            jnp.pad(p['bxz'] + p['bhz'], (0, dp - dout)),
            jnp.pad(p['bxr'] + p['bhr'], (0, dp - dout)),
            jnp.pad(p['bxh'] + p['bhh'], (0, dp - dout)),
        ])[None, :]
        inputs += [wx, wh, whh, b]
        in_specs += [
            pl.BlockSpec(wx.shape, lambda i: (0, 0)),
            pl.BlockSpec(wh.shape, lambda i: (0, 0)),
            pl.BlockSpec(whh.shape, lambda i: (0, 0)),
            pl.BlockSpec(b.shape, lambda i: (0, 0)),
        ]
    return pl.pallas_call(
        _gru_stack_body,
        grid=(N // _BN,),
        in_specs=in_specs,
        out_specs=pl.BlockSpec((T, _BN, 2), lambda i: (0, i, 0)),
        out_shape=jax.ShapeDtypeStruct((T, N, 2), jnp.float32),
        compiler_params=pltpu.CompilerParams(
            dimension_semantics=("parallel",)),
    )(*inputs)


# speedup vs baseline: 1.0698x; 1.0294x over previous
import jax
import jax.numpy as jnp
from jax.experimental import pallas as pl
from jax.experimental.pallas import tpu as pltpu

_DIMS = [(128, 256), (256, 128), (128, 64), (64, 32), (32, 2)]
_PAD_DIN = [128, 256, 128, 128, 128]
_PAD_DOUT = [256, 128, 128, 128, 128]
_BN = 1000


def _pad2(a, rows, cols):
    return jnp.pad(a, ((0, rows - a.shape[0]), (0, cols - a.shape[1])))


def _gru_stack_body(x_ref, *args):
    *w_refs, out_ref = args
    T = x_ref.shape[0]
    bn = x_ref.shape[1]
    nl = len(_DIMS)
    hs = [jnp.zeros((bn, dp), jnp.float32) for dp in _PAD_DOUT]
    for t in range(T):
        x = x_ref[t]
        for i in range(nl):
            wx, wh, whh, b = (r[...] for r in w_refs[4 * i:4 * i + 4])
            dp = _PAD_DOUT[i]
            h = hs[i]
            xw = jnp.dot(x, wx, preferred_element_type=jnp.float32) + b
            hw = jnp.dot(h, wh, preferred_element_type=jnp.float32)
            u_z = jnp.tanh(xw[:, :dp] + hw[:, :dp])
            u_r = jnp.tanh(xw[:, dp:2 * dp] + hw[:, dp:])
            h_tilde = jnp.tanh(
                xw[:, 2 * dp:]
                + jnp.dot(h * (1.0 + u_r), whh,
                          preferred_element_type=jnp.float32))
            h_new = 0.5 * ((h + h_tilde) + u_z * (h - h_tilde))
            if i > 0:
                h_new = jnp.maximum(h_new, 0.0)
            hs[i] = h_new
            x = h_new
        out_ref[t] = hs[-1][:, :2]


def kernel(graphs, edge_index, params):
    del edge_index
    T, N, F = graphs.shape
    inputs = [graphs]
    in_specs = [pl.BlockSpec((T, _BN, F), lambda i: (0, i, 0))]
    for i, p in enumerate(params):
        _, dout = _DIMS[i]
        dip, dp = _PAD_DIN[i], _PAD_DOUT[i]
        wx = jnp.concatenate(
            [_pad2(0.5 * p['Wxz'], dip, dp),
             _pad2(0.5 * p['Wxr'], dip, dp),
             _pad2(p['Wxh'], dip, dp)], axis=1)
        wh = jnp.concatenate(
            [_pad2(0.5 * p['Whz'], dp, dp),
             _pad2(0.5 * p['Whr'], dp, dp)], axis=1)
        whh = _pad2(0.5 * p['Whh'], dp, dp)
        b = jnp.concatenate([
            jnp.pad(0.5 * (p['bxz'] + p['bhz']), (0, dp - dout)),
            jnp.pad(0.5 * (p['bxr'] + p['bhr']), (0, dp - dout)),
            jnp.pad(p['bxh'] + p['bhh'], (0, dp - dout)),
        ])[None, :]
        inputs += [wx, wh, whh, b]
        in_specs += [
            pl.BlockSpec(wx.shape, lambda i: (0, 0)),
            pl.BlockSpec(wh.shape, lambda i: (0, 0)),
            pl.BlockSpec(whh.shape, lambda i: (0, 0)),
            pl.BlockSpec(b.shape, lambda i: (0, 0)),
        ]
    return pl.pallas_call(
        _gru_stack_body,
        grid=(N // _BN,),
        in_specs=in_specs,
        out_specs=pl.BlockSpec((T, _BN, 2), lambda i: (0, i, 0)),
        out_shape=jax.ShapeDtypeStruct((T, N, 2), jnp.float32),
        compiler_params=pltpu.CompilerParams(
            dimension_semantics=("parallel",)),
    )(*inputs)


# speedup vs baseline: 1.3094x; 1.2240x over previous
import jax
import jax.numpy as jnp
from jax import lax
from jax.experimental import pallas as pl
from jax.experimental.pallas import tpu as pltpu

_DIMS = [(128, 256), (256, 128), (128, 64), (64, 32), (32, 2)]
_PAD_DIN = [128, 256, 128, 64, 32]
_PAD_DOUT = [256, 128, 64, 32, 8]
_BN = 1024

_DN = (((1,), (1,)), ((), ()))


def _pad2(a, rows, cols):
    return jnp.pad(a, ((0, rows - a.shape[0]), (0, cols - a.shape[1])))


def _gru_stack_body(x_ref, *args):
    *w_refs, out_ref = args
    T = x_ref.shape[0]
    bn = x_ref.shape[2]
    nl = len(_DIMS)
    hs = [jnp.zeros((dp, bn), jnp.float32) for dp in _PAD_DOUT]
    for t in range(T):
        x = x_ref[t]
        for i in range(nl):
            wx, wh, whh, b = (r[...] for r in w_refs[4 * i:4 * i + 4])
            dp = _PAD_DOUT[i]
            h = hs[i]
            xw = jnp.dot(wx, x, preferred_element_type=jnp.float32) + b
            hw = jnp.dot(wh, h, preferred_element_type=jnp.float32)
            u_z = jnp.tanh(xw[:dp] + hw[:dp])
            u_r = jnp.tanh(xw[dp:2 * dp] + hw[dp:])
            h_tilde = jnp.tanh(
                xw[2 * dp:]
                + jnp.dot(whh, h * (1.0 + u_r),
                          preferred_element_type=jnp.float32))
            h_new = 0.5 * ((h + h_tilde) + u_z * (h - h_tilde))
            if i > 0:
                h_new = jnp.maximum(h_new, 0.0)
            hs[i] = h_new
            x = h_new
        out_ref[t] = hs[-1][:2]


def kernel(graphs, edge_index, params):
    del edge_index
    T, N, F = graphs.shape
    n_pad = (-N) % _BN
    graphs_t = jnp.transpose(graphs, (0, 2, 1))
    graphs_t = jnp.pad(graphs_t, ((0, 0), (0, 0), (0, n_pad)))
    np_tot = N + n_pad
    inputs = [graphs_t]
    in_specs = [pl.BlockSpec((T, F, _BN), lambda i: (0, 0, i))]
    for i, p in enumerate(params):
        _, dout = _DIMS[i]
        dip, dp = _PAD_DIN[i], _PAD_DOUT[i]
        wx = jnp.concatenate(
            [_pad2(0.5 * p['Wxz'].T, dp, dip),
             _pad2(0.5 * p['Wxr'].T, dp, dip),
             _pad2(p['Wxh'].T, dp, dip)], axis=0)
        wh = jnp.concatenate(
            [_pad2(0.5 * p['Whz'].T, dp, dp),
             _pad2(0.5 * p['Whr'].T, dp, dp)], axis=0)
        whh = _pad2(0.5 * p['Whh'].T, dp, dp)
        b = jnp.concatenate([
            jnp.pad(0.5 * (p['bxz'] + p['bhz']), (0, dp - dout)),
            jnp.pad(0.5 * (p['bxr'] + p['bhr']), (0, dp - dout)),
            jnp.pad(p['bxh'] + p['bhh'], (0, dp - dout)),
        ])[:, None]
        inputs += [wx, wh, whh, b]
        in_specs += [
            pl.BlockSpec(wx.shape, lambda i: (0, 0)),
            pl.BlockSpec(wh.shape, lambda i: (0, 0)),
            pl.BlockSpec(whh.shape, lambda i: (0, 0)),
            pl.BlockSpec(b.shape, lambda i: (0, 0)),
        ]
    out_t = pl.pallas_call(
        _gru_stack_body,
        grid=(np_tot // _BN,),
        in_specs=in_specs,
        out_specs=pl.BlockSpec((T, 2, _BN), lambda i: (0, 0, i)),
        out_shape=jax.ShapeDtypeStruct((T, 2, np_tot), jnp.float32),
        compiler_params=pltpu.CompilerParams(
            dimension_semantics=("parallel",)),
    )(*inputs)
    return jnp.transpose(out_t[:, :, :N], (0, 2, 1))


# speedup vs baseline: 1.3369x; 1.0210x over previous
import jax
import jax.numpy as jnp
from jax import lax
from jax.experimental import pallas as pl
from jax.experimental.pallas import tpu as pltpu

_DIMS = [(128, 256), (256, 128), (128, 64), (64, 32), (32, 2)]
_PAD_DIN = [128, 256, 128, 64, 32]
_PAD_DOUT = [256, 128, 64, 32, 8]
_BN = 1000

_DN = (((1,), (1,)), ((), ()))


def _pad2(a, rows, cols):
    return jnp.pad(a, ((0, rows - a.shape[0]), (0, cols - a.shape[1])))


def _gru_stack_body(x_ref, *args):
    *w_refs, out_ref = args
    T = x_ref.shape[0]
    bn = x_ref.shape[1]
    nl = len(_DIMS)
    hs = [jnp.zeros((dp, bn), jnp.float32) for dp in _PAD_DOUT]
    for t in range(T):
        x = x_ref[t]
        for i in range(nl):
            wx, wh, whh, b = (r[...] for r in w_refs[4 * i:4 * i + 4])
            dp = _PAD_DOUT[i]
            h = hs[i]
            if i == 0:
                xw = lax.dot_general(
                    wx, x, _DN, preferred_element_type=jnp.float32) + b
            else:
                xw = jnp.dot(
                    wx, x, preferred_element_type=jnp.float32) + b
            hw = jnp.dot(wh, h, preferred_element_type=jnp.float32)
            u_z = jnp.tanh(xw[:dp] + hw[:dp])
            u_r = jnp.tanh(xw[dp:2 * dp] + hw[dp:])
            h_tilde = jnp.tanh(
                xw[2 * dp:]
                + jnp.dot(whh, h * (1.0 + u_r),
                          preferred_element_type=jnp.float32))
            h_new = 0.5 * ((h + h_tilde) + u_z * (h - h_tilde))
            if i > 0:
                h_new = jnp.maximum(h_new, 0.0)
            hs[i] = h_new
            x = h_new
        out_ref[t] = hs[-1][:2].T


def kernel(graphs, edge_index, params):
    del edge_index
    T, N, F = graphs.shape
    inputs = [graphs]
    in_specs = [pl.BlockSpec((T, _BN, F), lambda i: (0, i, 0))]
    for i, p in enumerate(params):
        _, dout = _DIMS[i]
        dip, dp = _PAD_DIN[i], _PAD_DOUT[i]
        wx = jnp.concatenate(
            [_pad2(0.5 * p['Wxz'].T, dp, dip),
             _pad2(0.5 * p['Wxr'].T, dp, dip),
             _pad2(p['Wxh'].T, dp, dip)], axis=0)
        wh = jnp.concatenate(
            [_pad2(0.5 * p['Whz'].T, dp, dp),
             _pad2(0.5 * p['Whr'].T, dp, dp)], axis=0)
        whh = _pad2(0.5 * p['Whh'].T, dp, dp)
        b = jnp.concatenate([
            jnp.pad(0.5 * (p['bxz'] + p['bhz']), (0, dp - dout)),
            jnp.pad(0.5 * (p['bxr'] + p['bhr']), (0, dp - dout)),
            jnp.pad(p['bxh'] + p['bhh'], (0, dp - dout)),
        ])[:, None]
        inputs += [wx, wh, whh, b]
        in_specs += [
            pl.BlockSpec(wx.shape, lambda i: (0, 0)),
            pl.BlockSpec(wh.shape, lambda i: (0, 0)),
            pl.BlockSpec(whh.shape, lambda i: (0, 0)),
            pl.BlockSpec(b.shape, lambda i: (0, 0)),
        ]
    return pl.pallas_call(
        _gru_stack_body,
        grid=(N // _BN,),
        in_specs=in_specs,
        out_specs=pl.BlockSpec((T, _BN, 2), lambda i: (0, i, 0)),
        out_shape=jax.ShapeDtypeStruct((T, N, 2), jnp.float32),
        compiler_params=pltpu.CompilerParams(
            dimension_semantics=("parallel",)),
    )(*inputs)


# speedup vs baseline: 1.4337x; 1.0725x over previous
import jax
import jax.numpy as jnp
from jax import lax
from jax.experimental import pallas as pl
from jax.experimental.pallas import tpu as pltpu

_DIMS = [(128, 256), (256, 128), (128, 64), (64, 32), (32, 2)]
_PAD_DIN = [128, 256, 128, 64, 32]
_PAD_DOUT = [256, 128, 64, 32, 8]
_BN = 2000

_DN = (((1,), (1,)), ((), ()))


def _pad2(a, rows, cols):
    return jnp.pad(a, ((0, rows - a.shape[0]), (0, cols - a.shape[1])))


def _gru_stack_body(x_ref, *args):
    *w_refs, out_ref = args
    T = x_ref.shape[0]
    bn = x_ref.shape[1]
    nl = len(_DIMS)
    hs = [jnp.zeros((dp, bn), jnp.float32) for dp in _PAD_DOUT]
    for t in range(T):
        x = x_ref[t]
        for i in range(nl):
            wx, wh, whh, b = (r[...] for r in w_refs[4 * i:4 * i + 4])
            dp = _PAD_DOUT[i]
            h = hs[i]
            if i == 0:
                xw = lax.dot_general(
                    wx, x, _DN, preferred_element_type=jnp.float32) + b
            else:
                xw = jnp.dot(
                    wx, x, preferred_element_type=jnp.float32) + b
            hw = jnp.dot(wh, h, preferred_element_type=jnp.float32)
            u_z = jnp.tanh(xw[:dp] + hw[:dp])
            u_r = jnp.tanh(xw[dp:2 * dp] + hw[dp:])
            h_tilde = jnp.tanh(
                xw[2 * dp:]
                + jnp.dot(whh, h * (1.0 + u_r),
                          preferred_element_type=jnp.float32))
            h_new = 0.5 * ((h + h_tilde) + u_z * (h - h_tilde))
            if i > 0:
                h_new = jnp.maximum(h_new, 0.0)
            hs[i] = h_new
            x = h_new
        out_ref[t] = hs[-1][:2].T


def kernel(graphs, edge_index, params):
    del edge_index
    T, N, F = graphs.shape
    inputs = [graphs]
    in_specs = [pl.BlockSpec((T, _BN, F), lambda i: (0, i, 0))]
    for i, p in enumerate(params):
        _, dout = _DIMS[i]
        dip, dp = _PAD_DIN[i], _PAD_DOUT[i]
        wx = jnp.concatenate(
            [_pad2(0.5 * p['Wxz'].T, dp, dip),
             _pad2(0.5 * p['Wxr'].T, dp, dip),
             _pad2(p['Wxh'].T, dp, dip)], axis=0)
        wh = jnp.concatenate(
            [_pad2(0.5 * p['Whz'].T, dp, dp),
             _pad2(0.5 * p['Whr'].T, dp, dp)], axis=0)
        whh = _pad2(0.5 * p['Whh'].T, dp, dp)
        b = jnp.concatenate([
            jnp.pad(0.5 * (p['bxz'] + p['bhz']), (0, dp - dout)),
            jnp.pad(0.5 * (p['bxr'] + p['bhr']), (0, dp - dout)),
            jnp.pad(p['bxh'] + p['bhh'], (0, dp - dout)),
        ])[:, None]
        inputs += [wx, wh, whh, b]
        in_specs += [
            pl.BlockSpec(wx.shape, lambda i: (0, 0)),
            pl.BlockSpec(wh.shape, lambda i: (0, 0)),
            pl.BlockSpec(whh.shape, lambda i: (0, 0)),
            pl.BlockSpec(b.shape, lambda i: (0, 0)),
        ]
    return pl.pallas_call(
        _gru_stack_body,
        grid=(N // _BN,),
        in_specs=in_specs,
        out_specs=pl.BlockSpec((T, _BN, 2), lambda i: (0, i, 0)),
        out_shape=jax.ShapeDtypeStruct((T, N, 2), jnp.float32),
        compiler_params=pltpu.CompilerParams(
            dimension_semantics=("parallel",)),
    )(*inputs)


# speedup vs baseline: 1.5329x; 1.0691x over previous
import jax
import jax.numpy as jnp
from jax import lax
from jax.experimental import pallas as pl
from jax.experimental.pallas import tpu as pltpu

_DIMS = [(128, 256), (256, 128), (128, 64), (64, 32), (32, 2)]
_PAD_DIN = [128, 256, 128, 64, 32]
_PAD_DOUT = [256, 128, 64, 32, 8]
_BN = 2000

_DN = (((1,), (1,)), ((), ()))


def _pad2(a, rows, cols):
    return jnp.pad(a, ((0, rows - a.shape[0]), (0, cols - a.shape[1])))


def _gru_stack_body(x_ref, *args):
    *w_refs, out_ref = args
    T = x_ref.shape[0]
    bn = x_ref.shape[1]
    nl = len(_DIMS)
    hs = [jnp.zeros((dp, bn), jnp.float32) for dp in _PAD_DOUT]
    for t in range(T):
        x = x_ref[t]
        for i in range(nl):
            wx, wh, whh, b = (r[...] for r in w_refs[4 * i:4 * i + 4])
            dp = _PAD_DOUT[i]
            h = hs[i]
            if i == 0:
                xw = lax.dot_general(
                    wx, x.astype(jnp.bfloat16), _DN,
                    preferred_element_type=jnp.float32) + b
            else:
                xw = jnp.dot(
                    wx, x, preferred_element_type=jnp.float32) + b
            hw = jnp.dot(wh, h.astype(jnp.bfloat16),
                         preferred_element_type=jnp.float32)
            u_z = jnp.tanh(xw[:dp] + hw[:dp])
            u_r = jnp.tanh(xw[dp:2 * dp] + hw[dp:])
            h_tilde = jnp.tanh(
                xw[2 * dp:]
                + jnp.dot(whh, (h * (1.0 + u_r)).astype(jnp.bfloat16),
                          preferred_element_type=jnp.float32))
            h_new = 0.5 * ((h + h_tilde) + u_z * (h - h_tilde))
            if i > 0:
                h_new = jnp.maximum(h_new, 0.0)
            hs[i] = h_new
            x = h_new.astype(jnp.bfloat16)
        out_ref[t] = hs[-1][:2].T


def kernel(graphs, edge_index, params):
    del edge_index
    T, N, F = graphs.shape
    inputs = [graphs]
    in_specs = [pl.BlockSpec((T, _BN, F), lambda i: (0, i, 0))]
    for i, p in enumerate(params):
        _, dout = _DIMS[i]
        dip, dp = _PAD_DIN[i], _PAD_DOUT[i]
        wx = jnp.concatenate(
            [_pad2(0.5 * p['Wxz'].T, dp, dip),
             _pad2(0.5 * p['Wxr'].T, dp, dip),
             _pad2(p['Wxh'].T, dp, dip)], axis=0).astype(jnp.bfloat16)
        wh = jnp.concatenate(
            [_pad2(0.5 * p['Whz'].T, dp, dp),
             _pad2(0.5 * p['Whr'].T, dp, dp)], axis=0).astype(jnp.bfloat16)
        whh = _pad2(0.5 * p['Whh'].T, dp, dp).astype(jnp.bfloat16)
        b = jnp.concatenate([
            jnp.pad(0.5 * (p['bxz'] + p['bhz']), (0, dp - dout)),
            jnp.pad(0.5 * (p['bxr'] + p['bhr']), (0, dp - dout)),
            jnp.pad(p['bxh'] + p['bhh'], (0, dp - dout)),
        ])[:, None]
        inputs += [wx, wh, whh, b]
        in_specs += [
            pl.BlockSpec(wx.shape, lambda i: (0, 0)),
            pl.BlockSpec(wh.shape, lambda i: (0, 0)),
            pl.BlockSpec(whh.shape, lambda i: (0, 0)),
            pl.BlockSpec(b.shape, lambda i: (0, 0)),
        ]
    return pl.pallas_call(
        _gru_stack_body,
        grid=(N // _BN,),
        in_specs=in_specs,
        out_specs=pl.BlockSpec((T, _BN, 2), lambda i: (0, i, 0)),
        out_shape=jax.ShapeDtypeStruct((T, N, 2), jnp.float32),
        compiler_params=pltpu.CompilerParams(
            dimension_semantics=("parallel",)),
    )(*inputs)
